# Initial kernel scaffold; baseline (speedup 1.0000x reference)
import jax
import jax.numpy as jnp
from jax.experimental import pallas as pl


def kernel(x, params, batch_seq, batchsize):
    raise NotImplementedError("write your pallas kernel here")


# speedup vs baseline: 14.9418x; 14.9418x over previous
import functools

import jax
import jax.numpy as jnp
from jax.experimental import pallas as pl
from jax.experimental.pallas import tpu as pltpu

SIZE = 8
S = SIZE * SIZE
P = 448 // SIZE
N_FRAC = 0.25


def _rank_body(sel_ref, dist_ref, acc_ref, rcm_ref):
    s = sel_ref[:]
    m = jnp.max(s, axis=1, keepdims=True)
    e = jnp.exp(s - m)
    rcm_ref[:] = e / jnp.sum(e, axis=1, keepdims=True)
    acc_ref[:] = jnp.zeros_like(acc_ref)

    def body(i, _):
        a = rcm_ref[:] + dist_ref[i][None, :, :]
        m1 = jnp.max(a, axis=(1, 2), keepdims=True)
        masked = jnp.where(a == m1, -jnp.inf, a)
        m2 = jnp.max(masked, axis=(1, 2), keepdims=True)
        d = m1 - m2
        b = jnp.maximum(a - (m2 + N_FRAC * d), 0.0) / ((1.0 - N_FRAC) * d)
        bmax = jnp.max(b, axis=(1, 2), keepdims=True)
        b = b / bmax
        row = jnp.sum(b, axis=2, keepdims=True)
        col = jnp.sum(b, axis=1, keepdims=True)
        rcm_ref[:] = a - 100000.0 * (row + col)
        acc_ref[:] = acc_ref[:] + b
        return 0

    jax.lax.fori_loop(0, S, body, 0)


def _permute_body(x_ref, m_ref, o_ref):
    xw = x_ref[0, 0]
    x4 = xw.reshape(SIZE, P, SIZE, P)
    xb = x4.transpose(0, 2, 1, 3).reshape(S, P * P)
    mt = m_ref[0]
    y = jax.lax.dot_general(
        mt, xb, (((0,), (0,)), ((), ())), preferred_element_type=jnp.float32
    )
    y4 = y.reshape(SIZE, SIZE, P, P).transpose(0, 2, 1, 3)
    o_ref[0, 0] = y4.reshape(448, 448)


def _make_disturb():
    keys = jax.random.split(jax.random.key(42), S)
    return jax.vmap(lambda k: jax.random.normal(k, (S, S), dtype=jnp.float32))(
        keys
    ) * 1e-6


def kernel(x, params, batch_seq, batchsize):
    B, C = x.shape[0], x.shape[1]
    start = batch_seq * batchsize
    sel = jax.lax.dynamic_slice_in_dim(params, start, B, axis=0)
    disturb = _make_disturb()

    result_r = pl.pallas_call(
        _rank_body,
        out_shape=jax.ShapeDtypeStruct((B, S, S), jnp.float32),
        scratch_shapes=[pltpu.VMEM((B, S, S), jnp.float32)],
    )(sel, disturb)

    out = pl.pallas_call(
        _permute_body,
        grid=(B, C),
        in_specs=[
            pl.BlockSpec((1, 1, 448, 448), lambda b, c: (b, c, 0, 0)),
            pl.BlockSpec((1, S, S), lambda b, c: (B - 1 - b, 0, 0)),
        ],
        out_specs=pl.BlockSpec((1, 1, 448, 448), lambda b, c: (b, c, 0, 0)),
        out_shape=jax.ShapeDtypeStruct((B, C, 448, 448), jnp.float32),
    )(x, result_r)

    result_rev = result_r[::-1]
    result = jnp.stack([result_rev, result_rev, result_rev], axis=1)
    return out, result


# speedup vs baseline: 29.6224x; 1.9825x over previous
import jax
import jax.numpy as jnp
from jax.experimental import pallas as pl
from jax.experimental.pallas import tpu as pltpu

SIZE = 8
S = SIZE * SIZE
P = 448 // SIZE
NEG = -jnp.inf


def _rank_body(sel_ref, dist_ref, acc_ref, srcb_ref, rcm_ref, msk_ref):
    s = sel_ref[:]
    m = jnp.max(s, axis=1, keepdims=True)
    e = jnp.exp(s - m)
    rcm_ref[:] = e / jnp.sum(e, axis=1, keepdims=True)
    msk_ref[:] = jnp.zeros_like(msk_ref)
    acc_ref[:] = jnp.zeros_like(acc_ref)
    srcb_ref[:] = jnp.zeros_like(srcb_ref)

    riota = jax.lax.broadcasted_iota(jnp.int32, (1, S, S), 1)
    ciota = jax.lax.broadcasted_iota(jnp.int32, (1, S, S), 2)
    fiota = riota * S + ciota
    diota = jax.lax.broadcasted_iota(jnp.int32, (1, S), 1)

    def body(i, _):
        a = rcm_ref[:] + dist_ref[i][None, :, :]
        rcm_ref[:] = a
        w = a + msk_ref[:]
        m1 = jnp.max(w, axis=(1, 2), keepdims=True)
        eq = w == m1
        fi = jnp.min(jnp.where(eq, fiota, S * S), axis=(1, 2), keepdims=True)
        acc_ref[:] = acc_ref[:] + jnp.where(eq, 1.0, 0.0)
        rsel = fi // S
        csel = fi - rsel * S
        hit = (riota == rsel) | (ciota == csel)
        msk_ref[:] = msk_ref[:] + jnp.where(hit, NEG, 0.0)
        fi2 = jnp.squeeze(fi, -1)
        r2 = fi2 // S
        c2 = fi2 - r2 * S
        srcb_ref[:] = srcb_ref[:] + jnp.where(diota == c2, r2, 0)
        return 0

    jax.lax.fori_loop(0, S, body, 0)


def _gather_body(t_ref, x_ref, o_ref, blk_ref):
    b = pl.program_id(0)
    B = pl.num_programs(0)
    for sh in range(SIZE):
        slab = x_ref[0, :, sh * P:(sh + 1) * P, :]
        for sw in range(SIZE):
            blk_ref[sh * SIZE + sw] = slab[:, :, sw * P:(sw + 1) * P]
    for h in range(SIZE):
        cols = []
        for w in range(SIZE):
            k = t_ref[B - 1 - b, h * SIZE + w]
            cols.append(blk_ref[k])
        o_ref[0, :, h * P:(h + 1) * P, :] = jnp.concatenate(cols, axis=-1)


def _make_disturb():
    keys = jax.random.split(jax.random.key(42), S)
    return jax.vmap(lambda k: jax.random.normal(k, (S, S), dtype=jnp.float32))(
        keys
    ) * 1e-6


def kernel(x, params, batch_seq, batchsize):
    B, C = x.shape[0], x.shape[1]
    start = batch_seq * batchsize
    sel = jax.lax.dynamic_slice_in_dim(params, start, B, axis=0)
    disturb = _make_disturb()

    perm, srcb = pl.pallas_call(
        _rank_body,
        out_shape=(
            jax.ShapeDtypeStruct((B, S, S), jnp.float32),
            jax.ShapeDtypeStruct((B, S), jnp.int32),
        ),
        scratch_shapes=[
            pltpu.VMEM((B, S, S), jnp.float32),
            pltpu.VMEM((B, S, S), jnp.float32),
        ],
    )(sel, disturb)

    out = pl.pallas_call(
        _gather_body,
        grid_spec=pltpu.PrefetchScalarGridSpec(
            num_scalar_prefetch=1,
            grid=(B,),
            in_specs=[pl.BlockSpec((1, C, 448, 448), lambda b, f: (b, 0, 0, 0))],
            out_specs=pl.BlockSpec((1, C, 448, 448), lambda b, f: (b, 0, 0, 0)),
            scratch_shapes=[pltpu.VMEM((S, C, P, P), jnp.float32)],
        ),
        out_shape=jax.ShapeDtypeStruct((B, C, 448, 448), jnp.float32),
    )(srcb, x)

    result_rev = perm[::-1]
    result = jnp.stack([result_rev, result_rev, result_rev], axis=1)
    return out, result
